# Initial kernel scaffold; baseline (speedup 1.0000x reference)
import functools

import jax
import jax.numpy as jnp
from jax import lax
from jax.experimental import pallas as pl
from jax.experimental.pallas import tpu as pltpu
from jax.experimental.pallas import tpu_sc as plsc

NN = 10000
E = 320000
F = 128
DE = 16
H = 32
C = 8

NP = 10240
NC = 2
NS = 16
NW = NC * NS
CH = 128
NCH = 79
EPW = NCH * CH
EPAD = NW * EPW
AW = 48
ROWS_PER_TILE = NP // NS


def _prep_body(x_ref, w1_ref, a1_ref, h_ref, s_ref):
    h = jnp.dot(x_ref[...], w1_ref[...], preferred_element_type=jnp.float32)
    h_ref[...] = h
    s_ref[...] = jnp.dot(h, a1_ref[...], preferred_element_type=jnp.float32)


def _tc_prep(xpad, W1, A1):
    blk = 512
    grid = NP // blk
    return pl.pallas_call(
        _prep_body,
        grid=(grid,),
        in_specs=[
            pl.BlockSpec((blk, F), lambda i: (i, 0)),
            pl.BlockSpec((F, H), lambda i: (0, 0)),
            pl.BlockSpec((H, 8), lambda i: (0, 0)),
        ],
        out_specs=[
            pl.BlockSpec((blk, H), lambda i: (i, 0)),
            pl.BlockSpec((blk, 8), lambda i: (i, 0)),
        ],
        out_shape=[
            jax.ShapeDtypeStruct((NP, H), jnp.float32),
            jax.ShapeDtypeStruct((NP, 8), jnp.float32),
        ],
    )(xpad, W1, A1)


def _edge_body(ea_ref, m_ref, o_ref):
    o_ref[...] = jnp.dot(ea_ref[...], m_ref[...],
                         preferred_element_type=jnp.float32)


def _tc_edge_proj(edge_attr, M16):
    blk = 512
    grid = E // blk
    return pl.pallas_call(
        _edge_body,
        grid=(grid,),
        in_specs=[
            pl.BlockSpec((blk, DE), lambda i: (i, 0)),
            pl.BlockSpec((DE, 16), lambda i: (0, 0)),
        ],
        out_specs=pl.BlockSpec((blk, 16), lambda i: (i, 0)),
        out_shape=jax.ShapeDtypeStruct((E, 16), jnp.float32),
    )(edge_attr, M16)


def _combine_norm(a0, a1, s, hpre, bias):
    num = a0[:, :H] + a1[:, :H]
    wsum = a0[:, H] + a1[:, H]
    gsum = a0[:, H + 1] + a1[:, H + 1]
    deg = a0[:, H + 2] + a1[:, H + 2]
    gl = gsum / jnp.maximum(deg, 1.0)
    ln = s[:, 0] + s[:, 1] + gl
    ln = jnp.where(ln >= 0, ln, 0.2 * ln)
    wl = jnp.exp(ln)
    return (num + wl[:, None] * hpre) / (wsum + wl)[:, None] + bias


def _combine1_body(a0_ref, a1_ref, hpre_ref, s_ref, b_ref, w2_ref,
                   a2h_ref, a2g_ref, c2_ref, h1_ref, h2p_ref, s2_ref):
    h1 = _combine_norm(a0_ref[...], a1_ref[...], s_ref[...], hpre_ref[...],
                       b_ref[...])
    h2p = jnp.dot(h1, w2_ref[...], preferred_element_type=jnp.float32)
    s2 = (jnp.dot(h2p, a2h_ref[...], preferred_element_type=jnp.float32)
          + jnp.dot(h1, a2g_ref[...], preferred_element_type=jnp.float32)
          + c2_ref[...])
    h1_ref[...] = h1
    h2p_ref[...] = h2p
    s2_ref[...] = s2


def _tc_combine1(acc0, acc1, h1pre, S1, bias1, W2, A2h, A2g, C2):
    blk = 512
    grid = NP // blk
    full32 = pl.BlockSpec((H, H), lambda i: (0, 0))
    full328 = pl.BlockSpec((H, 8), lambda i: (0, 0))
    return pl.pallas_call(
        _combine1_body,
        grid=(grid,),
        in_specs=[
            pl.BlockSpec((blk, AW), lambda i: (i, 0)),
            pl.BlockSpec((blk, AW), lambda i: (i, 0)),
            pl.BlockSpec((blk, H), lambda i: (i, 0)),
            pl.BlockSpec((blk, 8), lambda i: (i, 0)),
            pl.BlockSpec((1, H), lambda i: (0, 0)),
            full32, full328, full328,
            pl.BlockSpec((1, 8), lambda i: (0, 0)),
        ],
        out_specs=[
            pl.BlockSpec((blk, H), lambda i: (i, 0)),
            pl.BlockSpec((blk, H), lambda i: (i, 0)),
            pl.BlockSpec((blk, 8), lambda i: (i, 0)),
        ],
        out_shape=[
            jax.ShapeDtypeStruct((NP, H), jnp.float32),
            jax.ShapeDtypeStruct((NP, H), jnp.float32),
            jax.ShapeDtypeStruct((NP, 8), jnp.float32),
        ],
    )(acc0, acc1, h1pre, S1, bias1, W2, A2h, A2g, C2)


def _combine2_body(a0_ref, a1_ref, hpre_ref, s_ref, b_ref, h1_ref,
                   pa2_ref, pa1_ref, pb2_ref, pb1_ref, cst_ref,
                   p_ref, q_ref):
    h2 = _combine_norm(a0_ref[...], a1_ref[...], s_ref[...], hpre_ref[...],
                       b_ref[...])
    h1 = h1_ref[...]
    p = (jnp.dot(h2, pa2_ref[...], preferred_element_type=jnp.float32)
         + jnp.dot(h1, pa1_ref[...], preferred_element_type=jnp.float32))
    q = (jnp.dot(h2, pb2_ref[...], preferred_element_type=jnp.float32)
         + jnp.dot(h1, pb1_ref[...], preferred_element_type=jnp.float32)
         + cst_ref[...])
    z = jnp.zeros_like(p)
    p_ref[...] = jnp.concatenate([p, z], axis=1)
    q_ref[...] = jnp.concatenate([q, z], axis=1)


def _tc_combine2(acc0, acc1, h2pre, S2, bias2, h1, PA2, PA1, PB2, PB1, CST):
    blk = 512
    grid = NP // blk
    full328 = pl.BlockSpec((H, 8), lambda i: (0, 0))
    return pl.pallas_call(
        _combine2_body,
        grid=(grid,),
        in_specs=[
            pl.BlockSpec((blk, AW), lambda i: (i, 0)),
            pl.BlockSpec((blk, AW), lambda i: (i, 0)),
            pl.BlockSpec((blk, H), lambda i: (i, 0)),
            pl.BlockSpec((blk, 8), lambda i: (i, 0)),
            pl.BlockSpec((1, H), lambda i: (0, 0)),
            pl.BlockSpec((blk, H), lambda i: (i, 0)),
            full328, full328, full328, full328,
            pl.BlockSpec((1, 8), lambda i: (0, 0)),
        ],
        out_specs=[
            pl.BlockSpec((blk, 16), lambda i: (i, 0)),
            pl.BlockSpec((blk, 16), lambda i: (i, 0)),
        ],
        out_shape=[
            jax.ShapeDtypeStruct((NP, 16), jnp.float32),
            jax.ShapeDtypeStruct((NP, 16), jnp.float32),
        ],
    )(acc0, acc1, h2pre, S2, bias2, h1, PA2, PA1, PB2, PB1, CST)


def _sweep_body(use_pab, src_hbm, dst_hbm, ge_hbm, hpre_hbm, as_hbm, ad_hbm,
                pa_hbm, pb_hbm, acc0_hbm, acc1_hbm,
                as_v, ad_v, pa_v, pb_v, src_v, dst_v, ge_v,
                rows_v, out_v, wg_v, acc_s, gsem):
    core = lax.axis_index("c")
    sid = lax.axis_index("s")
    wid = sid * NC + core

    pltpu.sync_copy(as_hbm, as_v)
    pltpu.sync_copy(ad_hbm, ad_v)
    if use_pab:
        pltpu.sync_copy(pa_hbm, pa_v)
        pltpu.sync_copy(pb_hbm, pb_v)
    pltpu.sync_copy(src_hbm.at[wid], src_v)
    pltpu.sync_copy(dst_hbm.at[wid], dst_v)
    pltpu.sync_copy(ge_hbm.at[wid], ge_v)

    zv = jnp.zeros((16,), jnp.float32)

    def _zbody(e, carry):
        out_v[e, pl.ds(0, 16)] = zv
        out_v[e, pl.ds(16, 16)] = zv
        out_v[e, pl.ds(32, 16)] = zv
        return carry

    lax.fori_loop(0, CH, _zbody, 0)
    for r in range(ROWS_PER_TILE // CH):
        pltpu.sync_copy(out_v,
                        acc_s.at[pl.ds(sid * ROWS_PER_TILE + r * CH, CH)])
    plsc.subcore_barrier()

    ii = jnp.arange(16, dtype=jnp.int32)
    oh0 = (ii == 0).astype(jnp.float32)
    oh1 = (ii == 1).astype(jnp.float32)
    oh2 = (ii == 2).astype(jnp.float32)

    def _chunk(j, carry):
        pltpu.async_copy(hpre_hbm.at[src_v.at[j]], rows_v, gsem).wait()
        for i in range(CH // 16):
            s16 = src_v[j, pl.ds(i * 16, 16)]
            d16 = dst_v[j, pl.ds(i * 16, 16)]
            g16 = ge_v[j, pl.ds(i * 16, 16)]
            if use_pab:
                g16 = (g16 + plsc.load_gather(pa_v, [s16])
                       + plsc.load_gather(pb_v, [d16]))
            l16 = (plsc.load_gather(as_v, [s16])
                   + plsc.load_gather(ad_v, [d16]) + g16)
            l16 = jnp.where(l16 >= 0, l16, 0.2 * l16)
            w16 = jnp.exp(l16)
            wg_v[0, pl.ds(i * 16, 16)] = w16
            wg_v[1, pl.ds(i * 16, 16)] = g16

        def _edge(e, c2):
            w = wg_v[0, e]
            g = wg_v[1, e]
            out_v[e, pl.ds(0, 16)] = rows_v[e, pl.ds(0, 16)] * w
            out_v[e, pl.ds(16, 16)] = rows_v[e, pl.ds(16, 16)] * w
            out_v[e, pl.ds(32, 16)] = oh0 * w + oh1 * g + oh2
            return c2

        lax.fori_loop(0, CH, _edge, 0)
        pltpu.sync_copy(out_v, acc_s.at[dst_v.at[j]], add=True)
        return carry

    lax.fori_loop(0, NCH, _chunk, 0)
    plsc.subcore_barrier()

    rows = pl.ds(sid * ROWS_PER_TILE, ROWS_PER_TILE)

    @pl.when(core == 0)
    def _():
        pltpu.sync_copy(acc_s.at[rows], acc0_hbm.at[rows])

    @pl.when(core == 1)
    def _():
        pltpu.sync_copy(acc_s.at[rows], acc1_hbm.at[rows])


def _sc_sweep(use_pab, src3, dst3, ge3, hpre, as_t, ad_t, pa_t, pb_t):
    mesh = plsc.VectorSubcoreMesh(core_axis_name="c", subcore_axis_name="s")
    fn = pl.kernel(
        functools.partial(_sweep_body, use_pab),
        mesh=mesh,
        out_type=[
            jax.ShapeDtypeStruct((NP, AW), jnp.float32),
            jax.ShapeDtypeStruct((NP, AW), jnp.float32),
        ],
        scratch_types=[
            pltpu.VMEM((NP,), jnp.float32),
            pltpu.VMEM((NP,), jnp.float32),
            pltpu.VMEM((NP,), jnp.float32),
            pltpu.VMEM((NP,), jnp.float32),
            pltpu.VMEM((NCH, CH), jnp.int32),
            pltpu.VMEM((NCH, CH), jnp.int32),
            pltpu.VMEM((NCH, CH), jnp.float32),
            pltpu.VMEM((CH, H), jnp.float32),
            pltpu.VMEM((CH, AW), jnp.float32),
            pltpu.VMEM((2, CH), jnp.float32),
            pltpu.VMEM_SHARED((NP, AW), jnp.float32),
            pltpu.SemaphoreType.DMA,
        ],
    )
    return fn(src3, dst3, ge3, hpre, as_t, ad_t, pa_t, pb_t)


def _out_body(src_hbm, dst_hbm, t_hbm, p_hbm, q_hbm, o_hbm,
              src_v, dst_v, t_v, p_v, q_v, gsem):
    core = lax.axis_index("c")
    sid = lax.axis_index("s")
    wid = sid * NC + core
    pltpu.sync_copy(src_hbm.at[wid], src_v)
    pltpu.sync_copy(dst_hbm.at[wid], dst_v)

    def _chunk(j, carry):
        pltpu.async_copy(p_hbm.at[src_v.at[j]], p_v, gsem).wait()
        pltpu.async_copy(q_hbm.at[dst_v.at[j]], q_v, gsem).wait()
        pltpu.sync_copy(t_hbm.at[wid, j], t_v)

        def _edge(e, c2):
            t_v[e, pl.ds(0, 16)] = (t_v[e, pl.ds(0, 16)]
                                    + p_v[e, pl.ds(0, 16)]
                                    + q_v[e, pl.ds(0, 16)])
            return c2

        lax.fori_loop(0, CH, _edge, 0)
        pltpu.sync_copy(t_v, o_hbm.at[wid, j])
        return carry

    lax.fori_loop(0, NCH, _chunk, 0)


def _sc_outpass(src3, dst3, t4, ptab, qtab):
    mesh = plsc.VectorSubcoreMesh(core_axis_name="c", subcore_axis_name="s")
    fn = pl.kernel(
        _out_body,
        mesh=mesh,
        out_type=jax.ShapeDtypeStruct((NW, NCH, CH, 16), jnp.float32),
        scratch_types=[
            pltpu.VMEM((NCH, CH), jnp.int32),
            pltpu.VMEM((NCH, CH), jnp.int32),
            pltpu.VMEM((CH, 16), jnp.float32),
            pltpu.VMEM((CH, 16), jnp.float32),
            pltpu.VMEM((CH, 16), jnp.float32),
            pltpu.SemaphoreType.DMA,
        ],
    )
    return fn(src3, dst3, t4, ptab, qtab)


def kernel(x, edge_index, edge_attr, W1, att_src1, att_dst1, Wedge1,
           att_edge1, bias1, Wel1, bel1, W2, att_src2, att_dst2, Wedge2,
           att_edge2, bias2, Wel2, bel2, Wc, bc):
    f32 = jnp.float32

    v1 = Wedge1 @ att_edge1
    v2 = Wedge2 @ att_edge2
    U2a = Wel1[:H] @ v2
    U2b = Wel1[H:2 * H] @ v2
    g2w = Wel1[2 * H:] @ v2
    c0 = bel1 @ v2
    Rm = Wel2[2 * H:] @ Wc
    PA2 = Wel2[:H] @ Wc
    PB2 = Wel2[H:2 * H] @ Wc
    PA1 = Wel1[:H] @ Rm
    PB1 = Wel1[H:2 * H] @ Rm
    S8 = Wel1[2 * H:] @ Rm
    cst = bel1 @ Rm + bel2 @ Wc + bc

    zc = jnp.zeros((H, 1), f32)
    A1 = jnp.concatenate(
        [att_src1[:, None], att_dst1[:, None]] + [zc] * 6, axis=1)
    A2h = jnp.concatenate(
        [att_src2[:, None], att_dst2[:, None]] + [zc] * 6, axis=1)
    A2g = jnp.concatenate(
        [jnp.zeros((H, 2), f32), U2a[:, None], U2b[:, None],
         jnp.zeros((H, 4), f32)], axis=1)
    C2 = jnp.zeros((1, 8), f32).at[0, 3].set(c0)
    M16 = jnp.concatenate(
        [v1[:, None], g2w[:, None], S8, jnp.zeros((DE, 6), f32)], axis=1)
    CST = cst[None, :]

    xpad = jnp.pad(x, ((0, NP - NN), (0, 0)))
    src = edge_index[0]
    dst = edge_index[1]
    src3 = jnp.pad(src, (0, EPAD - E)).reshape(NW, NCH, CH)
    dst3 = jnp.pad(dst, (0, EPAD - E),
                   constant_values=NN).reshape(NW, NCH, CH)

    h1pre, S1 = _tc_prep(xpad, W1, A1)
    Gd = _tc_edge_proj(edge_attr, M16)

    g13 = jnp.pad(Gd[:, 0], (0, EPAD - E)).reshape(NW, NCH, CH)
    g23 = jnp.pad(Gd[:, 1], (0, EPAD - E)).reshape(NW, NCH, CH)
    t4 = jnp.pad(Gd[:, 2:10],
                 ((0, EPAD - E), (0, 8))).reshape(NW, NCH, CH, 16)

    ztab = jnp.zeros((NP,), f32)

    acc0, acc1 = _sc_sweep(False, src3, dst3, g13, h1pre,
                           S1[:, 0], S1[:, 1], ztab, ztab)
    h1, h2pre, S2 = _tc_combine1(acc0, acc1, h1pre, S1, bias1[None, :],
                                 W2, A2h, A2g, C2)

    acc0b, acc1b = _sc_sweep(True, src3, dst3, g23, h2pre,
                             S2[:, 0], S2[:, 1], S2[:, 2], S2[:, 3])
    ptab, qtab = _tc_combine2(acc0b, acc1b, h2pre, S2, bias2[None, :],
                              h1, PA2, PA1, PB2, PB1, CST)

    o4 = _sc_outpass(src3, dst3, t4, ptab, qtab)
    return o4.reshape(EPAD, 16)[:E, :8]


# speedup vs baseline: 12.2308x; 12.2308x over previous
import functools

import jax
import jax.numpy as jnp
from jax import lax
from jax.experimental import pallas as pl
from jax.experimental.pallas import tpu as pltpu
from jax.experimental.pallas import tpu_sc as plsc

NN = 10000
E = 320000
F = 128
DE = 16
H = 32
C = 8

NP = 10240
NC = 2
NS = 16
NW = NC * NS
CH = 128
NCH = 79
EPW = NCH * CH
EPAD = NW * EPW
AW = 48
ROWS_PER_TILE = NP // NS


def _prep_body(x_ref, w1_ref, a1_ref, h_ref, s_ref):
    h = jnp.dot(x_ref[...], w1_ref[...], preferred_element_type=jnp.float32)
    h_ref[...] = h
    s_ref[...] = jnp.dot(h, a1_ref[...], preferred_element_type=jnp.float32)


def _tc_prep(xpad, W1, A1):
    blk = 512
    grid = NP // blk
    return pl.pallas_call(
        _prep_body,
        grid=(grid,),
        in_specs=[
            pl.BlockSpec((blk, F), lambda i: (i, 0)),
            pl.BlockSpec((F, H), lambda i: (0, 0)),
            pl.BlockSpec((H, 8), lambda i: (0, 0)),
        ],
        out_specs=[
            pl.BlockSpec((blk, H), lambda i: (i, 0)),
            pl.BlockSpec((blk, 8), lambda i: (i, 0)),
        ],
        out_shape=[
            jax.ShapeDtypeStruct((NP, H), jnp.float32),
            jax.ShapeDtypeStruct((NP, 8), jnp.float32),
        ],
    )(xpad, W1, A1)


def _edge_body(ea_ref, m_ref, o_ref):
    o_ref[...] = jnp.dot(ea_ref[...], m_ref[...],
                         preferred_element_type=jnp.float32)


def _tc_edge_proj(edge_attr, M16):
    blk = 512
    grid = E // blk
    return pl.pallas_call(
        _edge_body,
        grid=(grid,),
        in_specs=[
            pl.BlockSpec((blk, DE), lambda i: (i, 0)),
            pl.BlockSpec((DE, 16), lambda i: (0, 0)),
        ],
        out_specs=pl.BlockSpec((blk, 16), lambda i: (i, 0)),
        out_shape=jax.ShapeDtypeStruct((E, 16), jnp.float32),
    )(edge_attr, M16)


def _combine_norm(a0, a1, s, hpre, bias):
    num = a0[:, :H] + a1[:, :H]
    wsum = a0[:, H] + a1[:, H]
    gsum = a0[:, H + 1] + a1[:, H + 1]
    deg = a0[:, H + 2] + a1[:, H + 2]
    gl = gsum / jnp.maximum(deg, 1.0)
    ln = s[:, 0] + s[:, 1] + gl
    ln = jnp.where(ln >= 0, ln, 0.2 * ln)
    wl = jnp.exp(ln)
    return (num + wl[:, None] * hpre) / (wsum + wl)[:, None] + bias


def _combine1_body(a0_ref, a1_ref, hpre_ref, s_ref, b_ref, w2_ref,
                   a2h_ref, a2g_ref, c2_ref, h1_ref, h2p_ref, s2_ref):
    h1 = _combine_norm(a0_ref[...], a1_ref[...], s_ref[...], hpre_ref[...],
                       b_ref[...])
    h2p = jnp.dot(h1, w2_ref[...], preferred_element_type=jnp.float32)
    s2 = (jnp.dot(h2p, a2h_ref[...], preferred_element_type=jnp.float32)
          + jnp.dot(h1, a2g_ref[...], preferred_element_type=jnp.float32)
          + c2_ref[...])
    h1_ref[...] = h1
    h2p_ref[...] = h2p
    s2_ref[...] = s2


def _tc_combine1(acc0, acc1, h1pre, S1, bias1, W2, A2h, A2g, C2):
    blk = 512
    grid = NP // blk
    full32 = pl.BlockSpec((H, H), lambda i: (0, 0))
    full328 = pl.BlockSpec((H, 8), lambda i: (0, 0))
    return pl.pallas_call(
        _combine1_body,
        grid=(grid,),
        in_specs=[
            pl.BlockSpec((blk, AW), lambda i: (i, 0)),
            pl.BlockSpec((blk, AW), lambda i: (i, 0)),
            pl.BlockSpec((blk, H), lambda i: (i, 0)),
            pl.BlockSpec((blk, 8), lambda i: (i, 0)),
            pl.BlockSpec((1, H), lambda i: (0, 0)),
            full32, full328, full328,
            pl.BlockSpec((1, 8), lambda i: (0, 0)),
        ],
        out_specs=[
            pl.BlockSpec((blk, H), lambda i: (i, 0)),
            pl.BlockSpec((blk, H), lambda i: (i, 0)),
            pl.BlockSpec((blk, 8), lambda i: (i, 0)),
        ],
        out_shape=[
            jax.ShapeDtypeStruct((NP, H), jnp.float32),
            jax.ShapeDtypeStruct((NP, H), jnp.float32),
            jax.ShapeDtypeStruct((NP, 8), jnp.float32),
        ],
    )(acc0, acc1, h1pre, S1, bias1, W2, A2h, A2g, C2)


def _combine2_body(a0_ref, a1_ref, hpre_ref, s_ref, b_ref, h1_ref,
                   pa2_ref, pa1_ref, pb2_ref, pb1_ref, cst_ref,
                   p_ref, q_ref):
    h2 = _combine_norm(a0_ref[...], a1_ref[...], s_ref[...], hpre_ref[...],
                       b_ref[...])
    h1 = h1_ref[...]
    p = (jnp.dot(h2, pa2_ref[...], preferred_element_type=jnp.float32)
         + jnp.dot(h1, pa1_ref[...], preferred_element_type=jnp.float32))
    q = (jnp.dot(h2, pb2_ref[...], preferred_element_type=jnp.float32)
         + jnp.dot(h1, pb1_ref[...], preferred_element_type=jnp.float32)
         + cst_ref[...])
    z = jnp.zeros_like(p)
    p_ref[...] = jnp.concatenate([p, z], axis=1)
    q_ref[...] = jnp.concatenate([q, z], axis=1)


def _tc_combine2(acc0, acc1, h2pre, S2, bias2, h1, PA2, PA1, PB2, PB1, CST):
    blk = 512
    grid = NP // blk
    full328 = pl.BlockSpec((H, 8), lambda i: (0, 0))
    return pl.pallas_call(
        _combine2_body,
        grid=(grid,),
        in_specs=[
            pl.BlockSpec((blk, AW), lambda i: (i, 0)),
            pl.BlockSpec((blk, AW), lambda i: (i, 0)),
            pl.BlockSpec((blk, H), lambda i: (i, 0)),
            pl.BlockSpec((blk, 8), lambda i: (i, 0)),
            pl.BlockSpec((1, H), lambda i: (0, 0)),
            pl.BlockSpec((blk, H), lambda i: (i, 0)),
            full328, full328, full328, full328,
            pl.BlockSpec((1, 8), lambda i: (0, 0)),
        ],
        out_specs=[
            pl.BlockSpec((blk, 16), lambda i: (i, 0)),
            pl.BlockSpec((blk, 16), lambda i: (i, 0)),
        ],
        out_shape=[
            jax.ShapeDtypeStruct((NP, 16), jnp.float32),
            jax.ShapeDtypeStruct((NP, 16), jnp.float32),
        ],
    )(acc0, acc1, h2pre, S2, bias2, h1, PA2, PA1, PB2, PB1, CST)


def _sweep_body(use_pab, src_hbm, dst_hbm, ge_hbm, stab_hbm, dtab_hbm,
                ohs_hbm, acc0_hbm, acc1_hbm,
                src_v, dst_v, rows_v, drows_v, ge_v, out_v, ohs_v,
                acc_s, gsem):
    core = lax.axis_index("c")
    sid = lax.axis_index("s")
    wid = sid * NC + core

    pltpu.sync_copy(src_hbm.at[wid], src_v)
    pltpu.sync_copy(dst_hbm.at[wid], dst_v)

    zv = jnp.zeros((16,), jnp.float32)

    def _zbody(e, carry):
        out_v[e, pl.ds(0, 16)] = zv
        out_v[e, pl.ds(16, 16)] = zv
        out_v[e, pl.ds(32, 16)] = zv
        return carry

    lax.fori_loop(0, CH, _zbody, 0)
    for r in range(ROWS_PER_TILE // CH):
        pltpu.sync_copy(out_v,
                        acc_s.at[pl.ds(sid * ROWS_PER_TILE + r * CH, CH)])
    plsc.subcore_barrier()

    pltpu.sync_copy(ohs_hbm, ohs_v)
    oh0 = ohs_v[0, pl.ds(0, 16)]
    oh1 = ohs_v[1, pl.ds(0, 16)]
    oh2 = ohs_v[2, pl.ds(0, 16)]

    def _chunk(j, carry):
        cp1 = pltpu.async_copy(stab_hbm.at[src_v.at[j]], rows_v, gsem)
        cp2 = pltpu.async_copy(dtab_hbm.at[dst_v.at[j]], drows_v, gsem)
        pltpu.sync_copy(ge_hbm.at[wid, j], ge_v)
        cp1.wait()
        cp2.wait()
        for e in range(CH):
            g = ge_v[e, pl.ds(0, 16)]
            if use_pab:
                g = (g + rows_v[e, pl.ds(H + 16, 16)]
                     + drows_v[e, pl.ds(16, 16)])
            l = rows_v[e, pl.ds(H, 16)] + drows_v[e, pl.ds(0, 16)] + g
            l = jnp.maximum(l, 0.2 * l)
            w = jnp.exp(l)
            out_v[e, pl.ds(0, 16)] = rows_v[e, pl.ds(0, 16)] * w
            out_v[e, pl.ds(16, 16)] = rows_v[e, pl.ds(16, 16)] * w
            out_v[e, pl.ds(32, 16)] = oh0 * w + oh1 * g + oh2
        pltpu.sync_copy(out_v, acc_s.at[dst_v.at[j]], add=True)
        return carry

    lax.fori_loop(0, NCH, _chunk, 0)
    plsc.subcore_barrier()

    rows = pl.ds(sid * ROWS_PER_TILE, ROWS_PER_TILE)

    @pl.when(core == 0)
    def _():
        pltpu.sync_copy(acc_s.at[rows], acc0_hbm.at[rows])

    @pl.when(core == 1)
    def _():
        pltpu.sync_copy(acc_s.at[rows], acc1_hbm.at[rows])


def _sc_sweep(use_pab, src3, dst3, ge4, stab, dtab):
    sw = H + (32 if use_pab else 16)
    dw = 32 if use_pab else 16
    mesh = plsc.VectorSubcoreMesh(core_axis_name="c", subcore_axis_name="s")
    fn = pl.kernel(
        functools.partial(_sweep_body, use_pab),
        mesh=mesh,
        out_type=[
            jax.ShapeDtypeStruct((NP, AW), jnp.float32),
            jax.ShapeDtypeStruct((NP, AW), jnp.float32),
        ],
        scratch_types=[
            pltpu.VMEM((NCH, CH), jnp.int32),
            pltpu.VMEM((NCH, CH), jnp.int32),
            pltpu.VMEM((CH, sw), jnp.float32),
            pltpu.VMEM((CH, dw), jnp.float32),
            pltpu.VMEM((CH, 16), jnp.float32),
            pltpu.VMEM((CH, AW), jnp.float32),
            pltpu.VMEM((3, 16), jnp.float32),
            pltpu.VMEM_SHARED((NP, AW), jnp.float32),
            pltpu.SemaphoreType.DMA,
        ],
        compiler_params=pltpu.CompilerParams(use_tc_tiling_on_sc=False),
    )
    ohs = jnp.eye(3, 16, dtype=jnp.float32)
    return fn(src3, dst3, ge4, stab, dtab, ohs)


def _out_body(src_hbm, dst_hbm, t_hbm, p_hbm, q_hbm, o_hbm,
              src_v, dst_v, t_v, p_v, q_v, gsem):
    core = lax.axis_index("c")
    sid = lax.axis_index("s")
    wid = sid * NC + core
    pltpu.sync_copy(src_hbm.at[wid], src_v)
    pltpu.sync_copy(dst_hbm.at[wid], dst_v)

    def _chunk(j, carry):
        cp1 = pltpu.async_copy(p_hbm.at[src_v.at[j]], p_v, gsem)
        cp2 = pltpu.async_copy(q_hbm.at[dst_v.at[j]], q_v, gsem)
        pltpu.sync_copy(t_hbm.at[wid, j], t_v)
        cp1.wait()
        cp2.wait()

        def _edge(e, c2):
            t_v[e, pl.ds(0, 16)] = (t_v[e, pl.ds(0, 16)]
                                    + p_v[e, pl.ds(0, 16)]
                                    + q_v[e, pl.ds(0, 16)])
            return c2

        lax.fori_loop(0, CH, _edge, 0)
        pltpu.sync_copy(t_v, o_hbm.at[wid, j])
        return carry

    lax.fori_loop(0, NCH, _chunk, 0)


def _sc_outpass(src3, dst3, t4, ptab, qtab):
    mesh = plsc.VectorSubcoreMesh(core_axis_name="c", subcore_axis_name="s")
    fn = pl.kernel(
        _out_body,
        mesh=mesh,
        out_type=jax.ShapeDtypeStruct((NW, NCH, CH, 16), jnp.float32),
        scratch_types=[
            pltpu.VMEM((NCH, CH), jnp.int32),
            pltpu.VMEM((NCH, CH), jnp.int32),
            pltpu.VMEM((CH, 16), jnp.float32),
            pltpu.VMEM((CH, 16), jnp.float32),
            pltpu.VMEM((CH, 16), jnp.float32),
            pltpu.SemaphoreType.DMA,
        ],
        compiler_params=pltpu.CompilerParams(use_tc_tiling_on_sc=False),
    )
    return fn(src3, dst3, t4, ptab, qtab)


def kernel(x, edge_index, edge_attr, W1, att_src1, att_dst1, Wedge1,
           att_edge1, bias1, Wel1, bel1, W2, att_src2, att_dst2, Wedge2,
           att_edge2, bias2, Wel2, bel2, Wc, bc):
    f32 = jnp.float32

    v1 = Wedge1 @ att_edge1
    v2 = Wedge2 @ att_edge2
    U2a = Wel1[:H] @ v2
    U2b = Wel1[H:2 * H] @ v2
    g2w = Wel1[2 * H:] @ v2
    c0 = bel1 @ v2
    Rm = Wel2[2 * H:] @ Wc
    PA2 = Wel2[:H] @ Wc
    PB2 = Wel2[H:2 * H] @ Wc
    PA1 = Wel1[:H] @ Rm
    PB1 = Wel1[H:2 * H] @ Rm
    S8 = Wel1[2 * H:] @ Rm
    cst = bel1 @ Rm + bel2 @ Wc + bc

    zc = jnp.zeros((H, 1), f32)
    A1 = jnp.concatenate(
        [att_src1[:, None], att_dst1[:, None]] + [zc] * 6, axis=1)
    A2h = jnp.concatenate(
        [att_src2[:, None], att_dst2[:, None]] + [zc] * 6, axis=1)
    A2g = jnp.concatenate(
        [jnp.zeros((H, 2), f32), U2a[:, None], U2b[:, None],
         jnp.zeros((H, 4), f32)], axis=1)
    C2 = jnp.zeros((1, 8), f32).at[0, 3].set(c0)
    M16 = jnp.concatenate(
        [v1[:, None], g2w[:, None], S8, jnp.zeros((DE, 6), f32)], axis=1)
    CST = cst[None, :]

    xpad = jnp.pad(x, ((0, NP - NN), (0, 0)))
    src = edge_index[0]
    dst = edge_index[1]
    src3 = jnp.pad(src, (0, EPAD - E)).reshape(NW, NCH, CH)
    dst3 = jnp.pad(dst, (0, EPAD - E),
                   constant_values=NN).reshape(NW, NCH, CH)

    def splat(col):
        return jnp.broadcast_to(col[:, None], (NP, 16))

    def esplat(col):
        return jnp.broadcast_to(
            jnp.pad(col, (0, EPAD - E))[:, None],
            (EPAD, 16)).reshape(NW, NCH, CH, 16)

    h1pre, S1 = _tc_prep(xpad, W1, A1)
    Gd = _tc_edge_proj(edge_attr, M16)

    t4 = jnp.pad(Gd[:, 2:10],
                 ((0, EPAD - E), (0, 8))).reshape(NW, NCH, CH, 16)

    stab1 = jnp.concatenate([h1pre, splat(S1[:, 0])], axis=1)
    dtab1 = splat(S1[:, 1])
    acc0, acc1 = _sc_sweep(False, src3, dst3, esplat(Gd[:, 0]), stab1, dtab1)
    h1, h2pre, S2 = _tc_combine1(acc0, acc1, h1pre, S1, bias1[None, :],
                                 W2, A2h, A2g, C2)

    stab2 = jnp.concatenate(
        [h2pre, splat(S2[:, 0]), splat(S2[:, 2])], axis=1)
    dtab2 = jnp.concatenate(
        [splat(S2[:, 1]), splat(S2[:, 3])], axis=1)
    acc0b, acc1b = _sc_sweep(True, src3, dst3, esplat(Gd[:, 1]),
                             stab2, dtab2)
    ptab, qtab = _tc_combine2(acc0b, acc1b, h2pre, S2, bias2[None, :],
                              h1, PA2, PA1, PB2, PB1, CST)

    o4 = _sc_outpass(src3, dst3, t4, ptab, qtab)
    return o4.reshape(EPAD, 16)[:E, :8]


# speedup vs baseline: 23.3162x; 1.9064x over previous
import functools

import jax
import jax.numpy as jnp
from jax import lax
from jax.experimental import pallas as pl
from jax.experimental.pallas import tpu as pltpu
from jax.experimental.pallas import tpu_sc as plsc

NN = 10000
E = 320000
F = 128
DE = 16
H = 32
C = 8

NP = 10240
NC = 2
NS = 16
NW = NC * NS
CH = 128
NGC = E // CH
NCH = 79
EPW = NCH * CH
EPAD = NW * EPW
AW = 48
ROWS_PER_TILE = NP // NS


def _prep_body(x_ref, w1_ref, a1_ref, h_ref, s_ref):
    h = jnp.dot(x_ref[...], w1_ref[...], preferred_element_type=jnp.float32)
    h_ref[...] = h
    s_ref[...] = jnp.dot(h, a1_ref[...], preferred_element_type=jnp.float32)


def _tc_prep(xpad, W1, A1):
    blk = 512
    grid = NP // blk
    return pl.pallas_call(
        _prep_body,
        grid=(grid,),
        in_specs=[
            pl.BlockSpec((blk, F), lambda i: (i, 0)),
            pl.BlockSpec((F, H), lambda i: (0, 0)),
            pl.BlockSpec((H, 8), lambda i: (0, 0)),
        ],
        out_specs=[
            pl.BlockSpec((blk, H), lambda i: (i, 0)),
            pl.BlockSpec((blk, 8), lambda i: (i, 0)),
        ],
        out_shape=[
            jax.ShapeDtypeStruct((NP, H), jnp.float32),
            jax.ShapeDtypeStruct((NP, 8), jnp.float32),
        ],
    )(xpad, W1, A1)


def _edge_body(ea_ref, m_ref, o_ref):
    o_ref[...] = jnp.dot(ea_ref[...], m_ref[...],
                         preferred_element_type=jnp.float32)


def _tc_edge_proj(edge_attr, M16):
    blk = 12800
    grid = E // blk
    return pl.pallas_call(
        _edge_body,
        grid=(grid,),
        in_specs=[
            pl.BlockSpec((blk, DE), lambda i: (i, 0)),
            pl.BlockSpec((DE, 16), lambda i: (0, 0)),
        ],
        out_specs=pl.BlockSpec((blk, 16), lambda i: (i, 0)),
        out_shape=jax.ShapeDtypeStruct((E, 16), jnp.float32),
    )(edge_attr, M16)


def _combine_norm(a0, a1, s, hpre, bias):
    num = a0[:, :H] + a1[:, :H]
    wsum = a0[:, H] + a1[:, H]
    gsum = a0[:, H + 1] + a1[:, H + 1]
    deg = a0[:, H + 2] + a1[:, H + 2]
    gl = jnp.where(deg > 0, gsum / jnp.maximum(deg, 1.0) + s[:, 3], 0.0)
    ln = s[:, 0] + s[:, 1] + gl
    ln = jnp.where(ln >= 0, ln, 0.2 * ln)
    wl = jnp.exp(ln)
    return (num + wl[:, None] * hpre) / (wsum + wl)[:, None] + bias


def _combine1_body(a0_ref, a1_ref, hpre_ref, s_ref, b_ref, w2_ref,
                   a2h_ref, a2g_ref, c2_ref, h1_ref, h2p_ref, s2_ref):
    h1 = _combine_norm(a0_ref[...], a1_ref[...], s_ref[...], hpre_ref[...],
                       b_ref[...])
    h2p = jnp.dot(h1, w2_ref[...], preferred_element_type=jnp.float32)
    s2 = (jnp.dot(h2p, a2h_ref[...], preferred_element_type=jnp.float32)
          + jnp.dot(h1, a2g_ref[...], preferred_element_type=jnp.float32)
          + c2_ref[...])
    h1_ref[...] = h1
    h2p_ref[...] = h2p
    s2_ref[...] = s2


def _tc_combine1(acc0, acc1, h1pre, S1, bias1, W2, A2h, A2g, C2):
    blk = 512
    grid = NP // blk
    full32 = pl.BlockSpec((H, H), lambda i: (0, 0))
    full328 = pl.BlockSpec((H, 8), lambda i: (0, 0))
    return pl.pallas_call(
        _combine1_body,
        grid=(grid,),
        in_specs=[
            pl.BlockSpec((blk, AW), lambda i: (i, 0)),
            pl.BlockSpec((blk, AW), lambda i: (i, 0)),
            pl.BlockSpec((blk, H), lambda i: (i, 0)),
            pl.BlockSpec((blk, 8), lambda i: (i, 0)),
            pl.BlockSpec((1, H), lambda i: (0, 0)),
            full32, full328, full328,
            pl.BlockSpec((1, 8), lambda i: (0, 0)),
        ],
        out_specs=[
            pl.BlockSpec((blk, H), lambda i: (i, 0)),
            pl.BlockSpec((blk, H), lambda i: (i, 0)),
            pl.BlockSpec((blk, 8), lambda i: (i, 0)),
        ],
        out_shape=[
            jax.ShapeDtypeStruct((NP, H), jnp.float32),
            jax.ShapeDtypeStruct((NP, H), jnp.float32),
            jax.ShapeDtypeStruct((NP, 8), jnp.float32),
        ],
    )(acc0, acc1, h1pre, S1, bias1, W2, A2h, A2g, C2)


def _combine2_body(a0_ref, a1_ref, hpre_ref, s_ref, b_ref, h1_ref,
                   pa2_ref, pa1_ref, pb2_ref, pb1_ref, cst_ref,
                   p_ref, q_ref):
    h2 = _combine_norm(a0_ref[...], a1_ref[...], s_ref[...], hpre_ref[...],
                       b_ref[...])
    h1 = h1_ref[...]
    p = (jnp.dot(h2, pa2_ref[...], preferred_element_type=jnp.float32)
         + jnp.dot(h1, pa1_ref[...], preferred_element_type=jnp.float32))
    q = (jnp.dot(h2, pb2_ref[...], preferred_element_type=jnp.float32)
         + jnp.dot(h1, pb1_ref[...], preferred_element_type=jnp.float32)
         + cst_ref[...])
    z = jnp.zeros_like(p)
    p_ref[...] = jnp.concatenate([p, z], axis=1)
    q_ref[...] = jnp.concatenate([q, z], axis=1)


def _tc_combine2(acc0, acc1, h2pre, S2, bias2, h1, PA2, PA1, PB2, PB1, CST):
    blk = 512
    grid = NP // blk
    full328 = pl.BlockSpec((H, 8), lambda i: (0, 0))
    return pl.pallas_call(
        _combine2_body,
        grid=(grid,),
        in_specs=[
            pl.BlockSpec((blk, AW), lambda i: (i, 0)),
            pl.BlockSpec((blk, AW), lambda i: (i, 0)),
            pl.BlockSpec((blk, H), lambda i: (i, 0)),
            pl.BlockSpec((blk, 8), lambda i: (i, 0)),
            pl.BlockSpec((1, H), lambda i: (0, 0)),
            pl.BlockSpec((blk, H), lambda i: (i, 0)),
            full328, full328, full328, full328,
            pl.BlockSpec((1, 8), lambda i: (0, 0)),
        ],
        out_specs=[
            pl.BlockSpec((blk, 16), lambda i: (i, 0)),
            pl.BlockSpec((blk, 16), lambda i: (i, 0)),
        ],
        out_shape=[
            jax.ShapeDtypeStruct((NP, 16), jnp.float32),
            jax.ShapeDtypeStruct((NP, 16), jnp.float32),
        ],
    )(acc0, acc1, h2pre, S2, bias2, h1, PA2, PA1, PB2, PB1, CST)


def _sweep_body(use_pab, glane, src_hbm, dst_hbm, gd_hbm, stab_hbm, dtab_hbm,
                ohs_hbm, acc0_hbm, acc1_hbm,
                src_v, dst_v, rows_v, drows_v, ge_v, out_v, ohs_v,
                acc_s, gsem):
    core = lax.axis_index("c")
    sid = lax.axis_index("s")
    wid = sid * NC + core

    pltpu.sync_copy(src_hbm.at[wid], src_v)
    pltpu.sync_copy(dst_hbm.at[wid], dst_v)
    pltpu.sync_copy(ohs_hbm, ohs_v)
    oh0 = ohs_v[0, pl.ds(0, 16)]
    oh1 = ohs_v[1, pl.ds(0, 16)]
    oh2 = ohs_v[2, pl.ds(0, 16)]
    one = ohs_v[3, pl.ds(0, 16)]

    zv = jnp.zeros((16,), jnp.float32)

    def _zbody(e, carry):
        out_v[e, pl.ds(0, 16)] = zv
        out_v[e, pl.ds(16, 16)] = zv
        out_v[e, pl.ds(32, 16)] = zv
        return carry

    lax.fori_loop(0, CH, _zbody, 0)
    for r in range(ROWS_PER_TILE // CH):
        pltpu.sync_copy(out_v,
                        acc_s.at[pl.ds(sid * ROWS_PER_TILE + r * CH, CH)])
    plsc.subcore_barrier()

    def _chunk(j, carry):
        gc = wid * NCH + j

        @pl.when(gc < NGC)
        def _():
            cp1 = pltpu.async_copy(stab_hbm.at[src_v.at[j]], rows_v, gsem)
            cp2 = pltpu.async_copy(dtab_hbm.at[dst_v.at[j]], drows_v, gsem)
            pltpu.sync_copy(gd_hbm.at[gc], ge_v)
            cp1.wait()
            cp2.wait()
            for e in range(CH):
                g = one * ge_v[e, pl.ds(0, 16)][glane]
                a = g
                if use_pab:
                    a = g + rows_v[e, pl.ds(H + 16, 16)]
                l = rows_v[e, pl.ds(H, 16)] + drows_v[e, pl.ds(0, 16)] + g
                l = jnp.maximum(l, 0.2 * l)
                w = jnp.exp(l)
                out_v[e, pl.ds(0, 16)] = rows_v[e, pl.ds(0, 16)] * w
                out_v[e, pl.ds(16, 16)] = rows_v[e, pl.ds(16, 16)] * w
                out_v[e, pl.ds(32, 16)] = oh0 * w + oh1 * a + oh2
            pltpu.sync_copy(out_v, acc_s.at[dst_v.at[j]], add=True)

        return carry

    lax.fori_loop(0, NCH, _chunk, 0)
    plsc.subcore_barrier()

    rows = pl.ds(sid * ROWS_PER_TILE, ROWS_PER_TILE)

    @pl.when(core == 0)
    def _():
        pltpu.sync_copy(acc_s.at[rows], acc0_hbm.at[rows])

    @pl.when(core == 1)
    def _():
        pltpu.sync_copy(acc_s.at[rows], acc1_hbm.at[rows])


def _sc_sweep(use_pab, glane, src3, dst3, gd3, stab, dtab, ohs):
    sw = H + (32 if use_pab else 16)
    mesh = plsc.VectorSubcoreMesh(core_axis_name="c", subcore_axis_name="s")
    fn = pl.kernel(
        functools.partial(_sweep_body, use_pab, glane),
        mesh=mesh,
        out_type=[
            jax.ShapeDtypeStruct((NP, AW), jnp.float32),
            jax.ShapeDtypeStruct((NP, AW), jnp.float32),
        ],
        scratch_types=[
            pltpu.VMEM((NCH, CH), jnp.int32),
            pltpu.VMEM((NCH, CH), jnp.int32),
            pltpu.VMEM((CH, sw), jnp.float32),
            pltpu.VMEM((CH, 16), jnp.float32),
            pltpu.VMEM((CH, 16), jnp.float32),
            pltpu.VMEM((CH, AW), jnp.float32),
            pltpu.VMEM((4, 16), jnp.float32),
            pltpu.VMEM_SHARED((NP, AW), jnp.float32),
            pltpu.SemaphoreType.DMA,
        ],
        compiler_params=pltpu.CompilerParams(use_tc_tiling_on_sc=False),
    )
    return fn(src3, dst3, gd3, stab, dtab, ohs)


def _out_body(src_hbm, dst_hbm, gd_hbm, p_hbm, q_hbm, o_hbm,
              src_v, dst_v, t_v, p_v, q_v, gsem):
    core = lax.axis_index("c")
    sid = lax.axis_index("s")
    wid = sid * NC + core
    pltpu.sync_copy(src_hbm.at[wid], src_v)
    pltpu.sync_copy(dst_hbm.at[wid], dst_v)

    def _chunk(j, carry):
        gc = wid * NCH + j

        @pl.when(gc < NGC)
        def _():
            cp1 = pltpu.async_copy(p_hbm.at[src_v.at[j]], p_v, gsem)
            cp2 = pltpu.async_copy(q_hbm.at[dst_v.at[j]], q_v, gsem)
            pltpu.sync_copy(gd_hbm.at[gc], t_v)
            cp1.wait()
            cp2.wait()

            def _edge(e, c2):
                t_v[e, pl.ds(0, 16)] = (t_v[e, pl.ds(0, 16)]
                                        + p_v[e, pl.ds(0, 16)]
                                        + q_v[e, pl.ds(0, 16)])
                return c2

            lax.fori_loop(0, CH, _edge, 0)
            pltpu.sync_copy(t_v, o_hbm.at[gc])

        return carry

    lax.fori_loop(0, NCH, _chunk, 0)


def _sc_outpass(src3, dst3, gd3, ptab, qtab):
    mesh = plsc.VectorSubcoreMesh(core_axis_name="c", subcore_axis_name="s")
    fn = pl.kernel(
        _out_body,
        mesh=mesh,
        out_type=jax.ShapeDtypeStruct((NGC, CH, 16), jnp.float32),
        scratch_types=[
            pltpu.VMEM((NCH, CH), jnp.int32),
            pltpu.VMEM((NCH, CH), jnp.int32),
            pltpu.VMEM((CH, 16), jnp.float32),
            pltpu.VMEM((CH, 16), jnp.float32),
            pltpu.VMEM((CH, 16), jnp.float32),
            pltpu.SemaphoreType.DMA,
        ],
        compiler_params=pltpu.CompilerParams(use_tc_tiling_on_sc=False),
    )
    return fn(src3, dst3, gd3, ptab, qtab)


def kernel(x, edge_index, edge_attr, W1, att_src1, att_dst1, Wedge1,
           att_edge1, bias1, Wel1, bel1, W2, att_src2, att_dst2, Wedge2,
           att_edge2, bias2, Wel2, bel2, Wc, bc):
    f32 = jnp.float32

    v1 = Wedge1 @ att_edge1
    v2 = Wedge2 @ att_edge2
    U2a = Wel1[:H] @ v2
    U2b = Wel1[H:2 * H] @ v2
    g2w = Wel1[2 * H:] @ v2
    c0 = bel1 @ v2
    Rm = Wel2[2 * H:] @ Wc
    PA2 = Wel2[:H] @ Wc
    PB2 = Wel2[H:2 * H] @ Wc
    PA1 = Wel1[:H] @ Rm
    PB1 = Wel1[H:2 * H] @ Rm
    S8 = Wel1[2 * H:] @ Rm
    cst = bel1 @ Rm + bel2 @ Wc + bc

    zc = jnp.zeros((H, 1), f32)
    A1 = jnp.concatenate(
        [att_src1[:, None], att_dst1[:, None]] + [zc] * 6, axis=1)
    A2h = jnp.concatenate(
        [att_src2[:, None], att_dst2[:, None]] + [zc] * 6, axis=1)
    A2g = jnp.concatenate(
        [jnp.zeros((H, 2), f32), U2a[:, None], U2b[:, None],
         jnp.zeros((H, 4), f32)], axis=1)
    C2 = jnp.zeros((1, 8), f32).at[0, 3].set(c0)
    M16 = jnp.concatenate(
        [S8, v1[:, None], g2w[:, None], jnp.zeros((DE, 6), f32)], axis=1)
    CST = cst[None, :]
    ohs = jnp.concatenate(
        [jnp.eye(3, 16, dtype=f32), jnp.ones((1, 16), f32)], axis=0)

    xpad = jnp.pad(x, ((0, NP - NN), (0, 0)))
    src = edge_index[0]
    dst = edge_index[1]
    src3 = jnp.pad(src, (0, EPAD - E)).reshape(NW, NCH, CH)
    dst3 = jnp.pad(dst, (0, EPAD - E),
                   constant_values=NN).reshape(NW, NCH, CH)

    def splat(col):
        return jnp.broadcast_to(col[:, None], (NP, 16))

    h1pre, S1 = _tc_prep(xpad, W1, A1)
    Gd = _tc_edge_proj(edge_attr, M16)
    gd3 = Gd.reshape(NGC, CH, 16)

    stab1 = jnp.concatenate([h1pre, splat(S1[:, 0])], axis=1)
    dtab1 = splat(S1[:, 1])
    acc0, acc1 = _sc_sweep(False, 8, src3, dst3, gd3, stab1, dtab1, ohs)
    h1, h2pre, S2 = _tc_combine1(acc0, acc1, h1pre, S1, bias1[None, :],
                                 W2, A2h, A2g, C2)

    stab2 = jnp.concatenate(
        [h2pre, splat(S2[:, 0] + S2[:, 2]), splat(S2[:, 2])], axis=1)
    dtab2 = splat(S2[:, 1] + S2[:, 3])
    acc0b, acc1b = _sc_sweep(True, 9, src3, dst3, gd3, stab2, dtab2, ohs)
    ptab, qtab = _tc_combine2(acc0b, acc1b, h2pre, S2, bias2[None, :],
                              h1, PA2, PA1, PB2, PB1, CST)

    o3 = _sc_outpass(src3, dst3, gd3, ptab, qtab)
    return o3.reshape(E, 16)[:, :8]


# speedup vs baseline: 27.9434x; 1.1985x over previous
import functools

import jax
import jax.numpy as jnp
from jax import lax
from jax.experimental import pallas as pl
from jax.experimental.pallas import tpu as pltpu
from jax.experimental.pallas import tpu_sc as plsc

NN = 10000
E = 320000
F = 128
DE = 16
H = 32
C = 8

NP = 10240
NC = 2
NS = 16
NW = NC * NS
CH = 128
NGC = E // CH
NCH = 79
EPW = NCH * CH
EPAD = NW * EPW
AW = 48
ROWS_PER_TILE = NP // NS


def _prep_body(x_ref, w1_ref, a1_ref, h_ref, s_ref):
    h = jnp.dot(x_ref[...], w1_ref[...], preferred_element_type=jnp.float32)
    h_ref[...] = h
    s_ref[...] = jnp.dot(h, a1_ref[...], preferred_element_type=jnp.float32)


def _tc_prep(x, W1, A1):
    blk = 2000
    grid = NN // blk
    return pl.pallas_call(
        _prep_body,
        grid=(grid,),
        in_specs=[
            pl.BlockSpec((blk, F), lambda i: (i, 0)),
            pl.BlockSpec((F, H), lambda i: (0, 0)),
            pl.BlockSpec((H, 8), lambda i: (0, 0)),
        ],
        out_specs=[
            pl.BlockSpec((blk, H), lambda i: (i, 0)),
            pl.BlockSpec((blk, 8), lambda i: (i, 0)),
        ],
        out_shape=[
            jax.ShapeDtypeStruct((NN, H), jnp.float32),
            jax.ShapeDtypeStruct((NN, 8), jnp.float32),
        ],
    )(x, W1, A1)


def _edge_body(ea_ref, m_ref, o_ref):
    o_ref[...] = jnp.dot(ea_ref[...], m_ref[...],
                         preferred_element_type=jnp.float32)


def _tc_edge_proj(ea8, Wbd):
    rows = E // 8
    blk = 8000
    grid = rows // blk
    return pl.pallas_call(
        _edge_body,
        grid=(grid,),
        in_specs=[
            pl.BlockSpec((blk, 128), lambda i: (i, 0)),
            pl.BlockSpec((128, 128), lambda i: (0, 0)),
        ],
        out_specs=pl.BlockSpec((blk, 128), lambda i: (i, 0)),
        out_shape=jax.ShapeDtypeStruct((rows, 128), jnp.float32),
    )(ea8, Wbd)


def _combine_norm(a0, a1, s, hpre, bias):
    num = a0[:, :H] + a1[:, :H]
    wsum = a0[:, H] + a1[:, H]
    gsum = a0[:, H + 1] + a1[:, H + 1]
    deg = a0[:, H + 2] + a1[:, H + 2]
    gl = jnp.where(deg > 0, gsum / jnp.maximum(deg, 1.0) + s[:, 3], 0.0)
    ln = s[:, 0] + s[:, 1] + gl
    ln = jnp.where(ln >= 0, ln, 0.2 * ln)
    wl = jnp.exp(ln)
    return (num + wl[:, None] * hpre) / (wsum + wl)[:, None] + bias


def _combine1_body(a0_ref, a1_ref, hpre_ref, s_ref, b_ref, w2_ref,
                   a2h_ref, a2g_ref, c2_ref, h1_ref, h2p_ref, s2_ref):
    h1 = _combine_norm(a0_ref[...], a1_ref[...], s_ref[...], hpre_ref[...],
                       b_ref[...])
    h2p = jnp.dot(h1, w2_ref[...], preferred_element_type=jnp.float32)
    s2 = (jnp.dot(h2p, a2h_ref[...], preferred_element_type=jnp.float32)
          + jnp.dot(h1, a2g_ref[...], preferred_element_type=jnp.float32)
          + c2_ref[...])
    h1_ref[...] = h1
    h2p_ref[...] = h2p
    s2_ref[...] = s2


def _tc_combine1(acc0, acc1, h1pre, S1, bias1, W2, A2h, A2g, C2):
    blk = 2000
    grid = NN // blk
    full32 = pl.BlockSpec((H, H), lambda i: (0, 0))
    full328 = pl.BlockSpec((H, 8), lambda i: (0, 0))
    return pl.pallas_call(
        _combine1_body,
        grid=(grid,),
        in_specs=[
            pl.BlockSpec((blk, AW), lambda i: (i, 0)),
            pl.BlockSpec((blk, AW), lambda i: (i, 0)),
            pl.BlockSpec((blk, H), lambda i: (i, 0)),
            pl.BlockSpec((blk, 8), lambda i: (i, 0)),
            pl.BlockSpec((1, H), lambda i: (0, 0)),
            full32, full328, full328,
            pl.BlockSpec((1, 8), lambda i: (0, 0)),
        ],
        out_specs=[
            pl.BlockSpec((blk, H), lambda i: (i, 0)),
            pl.BlockSpec((blk, H), lambda i: (i, 0)),
            pl.BlockSpec((blk, 8), lambda i: (i, 0)),
        ],
        out_shape=[
            jax.ShapeDtypeStruct((NN, H), jnp.float32),
            jax.ShapeDtypeStruct((NN, H), jnp.float32),
            jax.ShapeDtypeStruct((NN, 8), jnp.float32),
        ],
    )(acc0, acc1, h1pre, S1, bias1, W2, A2h, A2g, C2)


def _combine2_body(a0_ref, a1_ref, hpre_ref, s_ref, b_ref, h1_ref,
                   pa2_ref, pa1_ref, pb2_ref, pb1_ref, cst_ref,
                   p_ref, q_ref):
    h2 = _combine_norm(a0_ref[...], a1_ref[...], s_ref[...], hpre_ref[...],
                       b_ref[...])
    h1 = h1_ref[...]
    p = (jnp.dot(h2, pa2_ref[...], preferred_element_type=jnp.float32)
         + jnp.dot(h1, pa1_ref[...], preferred_element_type=jnp.float32))
    q = (jnp.dot(h2, pb2_ref[...], preferred_element_type=jnp.float32)
         + jnp.dot(h1, pb1_ref[...], preferred_element_type=jnp.float32)
         + cst_ref[...])
    z = jnp.zeros_like(p)
    p_ref[...] = jnp.concatenate([p, z], axis=1)
    q_ref[...] = jnp.concatenate([q, z], axis=1)


def _tc_combine2(acc0, acc1, h2pre, S2, bias2, h1, PA2, PA1, PB2, PB1, CST):
    blk = 2000
    grid = NN // blk
    full328 = pl.BlockSpec((H, 8), lambda i: (0, 0))
    return pl.pallas_call(
        _combine2_body,
        grid=(grid,),
        in_specs=[
            pl.BlockSpec((blk, AW), lambda i: (i, 0)),
            pl.BlockSpec((blk, AW), lambda i: (i, 0)),
            pl.BlockSpec((blk, H), lambda i: (i, 0)),
            pl.BlockSpec((blk, 8), lambda i: (i, 0)),
            pl.BlockSpec((1, H), lambda i: (0, 0)),
            pl.BlockSpec((blk, H), lambda i: (i, 0)),
            full328, full328, full328, full328,
            pl.BlockSpec((1, 8), lambda i: (0, 0)),
        ],
        out_specs=[
            pl.BlockSpec((blk, 16), lambda i: (i, 0)),
            pl.BlockSpec((blk, 16), lambda i: (i, 0)),
        ],
        out_shape=[
            jax.ShapeDtypeStruct((NN, 16), jnp.float32),
            jax.ShapeDtypeStruct((NN, 16), jnp.float32),
        ],
    )(acc0, acc1, h2pre, S2, bias2, h1, PA2, PA1, PB2, PB1, CST)


def _sweep_body(use_pab, glane, src_hbm, dst_hbm, gd_hbm, stab_hbm, dtab_hbm,
                ohs_hbm, acc0_hbm, acc1_hbm,
                src_v, dst_v, rows_v, drows_v, ge_v, out_v, ohs_v,
                acc_s, gsem):
    core = lax.axis_index("c")
    sid = lax.axis_index("s")
    wid = sid * NC + core

    pltpu.sync_copy(src_hbm.at[wid], src_v)
    pltpu.sync_copy(dst_hbm.at[wid], dst_v)
    pltpu.sync_copy(ohs_hbm, ohs_v)
    oh0 = ohs_v[0, pl.ds(0, 16)]
    oh1 = ohs_v[1, pl.ds(0, 16)]
    oh2 = ohs_v[2, pl.ds(0, 16)]
    one = ohs_v[3, pl.ds(0, 16)]

    zv = jnp.zeros((16,), jnp.float32)

    def _zbody(e, carry):
        out_v[e, pl.ds(0, 16)] = zv
        out_v[e, pl.ds(16, 16)] = zv
        out_v[e, pl.ds(32, 16)] = zv
        return carry

    lax.fori_loop(0, CH, _zbody, 0)
    for r in range(ROWS_PER_TILE // CH):
        pltpu.sync_copy(out_v,
                        acc_s.at[pl.ds(sid * ROWS_PER_TILE + r * CH, CH)])
    plsc.subcore_barrier()

    def _chunk(j, carry):
        gc = wid * NCH + j

        @pl.when(gc < NGC)
        def _():
            cp1 = pltpu.async_copy(stab_hbm.at[src_v.at[j]], rows_v, gsem)
            cp2 = pltpu.async_copy(dtab_hbm.at[dst_v.at[j]], drows_v, gsem)
            pltpu.sync_copy(gd_hbm.at[gc], ge_v)
            cp1.wait()
            cp2.wait()
            for e in range(CH):
                g = one * ge_v[e, pl.ds(0, 16)][glane]
                a = g
                if use_pab:
                    a = g + rows_v[e, pl.ds(H + 16, 16)]
                l = rows_v[e, pl.ds(H, 16)] + drows_v[e, pl.ds(0, 16)] + g
                l = jnp.maximum(l, 0.2 * l)
                w = jnp.exp(l)
                out_v[e, pl.ds(0, 16)] = rows_v[e, pl.ds(0, 16)] * w
                out_v[e, pl.ds(16, 16)] = rows_v[e, pl.ds(16, 16)] * w
                out_v[e, pl.ds(32, 16)] = oh0 * w + oh1 * a + oh2
            pltpu.sync_copy(out_v, acc_s.at[dst_v.at[j]], add=True)

        return carry

    lax.fori_loop(0, NCH, _chunk, 0)
    plsc.subcore_barrier()

    rows = pl.ds(sid * ROWS_PER_TILE, ROWS_PER_TILE)

    @pl.when(core == 0)
    def _():
        pltpu.sync_copy(acc_s.at[rows], acc0_hbm.at[rows])

    @pl.when(core == 1)
    def _():
        pltpu.sync_copy(acc_s.at[rows], acc1_hbm.at[rows])


def _sc_sweep(use_pab, glane, src3, dst3, gd3, stab, dtab, ohs):
    sw = H + (32 if use_pab else 16)
    mesh = plsc.VectorSubcoreMesh(core_axis_name="c", subcore_axis_name="s")
    fn = pl.kernel(
        functools.partial(_sweep_body, use_pab, glane),
        mesh=mesh,
        out_type=[
            jax.ShapeDtypeStruct((NP, AW), jnp.float32),
            jax.ShapeDtypeStruct((NP, AW), jnp.float32),
        ],
        scratch_types=[
            pltpu.VMEM((NCH, CH), jnp.int32),
            pltpu.VMEM((NCH, CH), jnp.int32),
            pltpu.VMEM((CH, sw), jnp.float32),
            pltpu.VMEM((CH, 16), jnp.float32),
            pltpu.VMEM((CH, 16), jnp.float32),
            pltpu.VMEM((CH, AW), jnp.float32),
            pltpu.VMEM((4, 16), jnp.float32),
            pltpu.VMEM_SHARED((NP, AW), jnp.float32),
            pltpu.SemaphoreType.DMA,
        ],
        compiler_params=pltpu.CompilerParams(use_tc_tiling_on_sc=False),
    )
    return fn(src3, dst3, gd3, stab, dtab, ohs)


def _out_body(src_hbm, dst_hbm, gd_hbm, p_hbm, q_hbm, o_hbm,
              src_v, dst_v, t_v, p_v, q_v, gsem):
    core = lax.axis_index("c")
    sid = lax.axis_index("s")
    wid = sid * NC + core
    pltpu.sync_copy(src_hbm.at[wid], src_v)
    pltpu.sync_copy(dst_hbm.at[wid], dst_v)

    def _chunk(j, carry):
        gc = wid * NCH + j

        @pl.when(gc < NGC)
        def _():
            cp1 = pltpu.async_copy(p_hbm.at[src_v.at[j]], p_v, gsem)
            cp2 = pltpu.async_copy(q_hbm.at[dst_v.at[j]], q_v, gsem)
            pltpu.sync_copy(gd_hbm.at[gc], t_v)
            cp1.wait()
            cp2.wait()

            def _edge(e, c2):
                t_v[e, pl.ds(0, 16)] = (t_v[e, pl.ds(0, 16)]
                                        + p_v[e, pl.ds(0, 16)]
                                        + q_v[e, pl.ds(0, 16)])
                return c2

            lax.fori_loop(0, CH, _edge, 0)
            pltpu.sync_copy(t_v, o_hbm.at[gc])

        return carry

    lax.fori_loop(0, NCH, _chunk, 0)


def _sc_outpass(src3, dst3, gd3, ptab, qtab):
    mesh = plsc.VectorSubcoreMesh(core_axis_name="c", subcore_axis_name="s")
    fn = pl.kernel(
        _out_body,
        mesh=mesh,
        out_type=jax.ShapeDtypeStruct((NGC, CH, 16), jnp.float32),
        scratch_types=[
            pltpu.VMEM((NCH, CH), jnp.int32),
            pltpu.VMEM((NCH, CH), jnp.int32),
            pltpu.VMEM((CH, 16), jnp.float32),
            pltpu.VMEM((CH, 16), jnp.float32),
            pltpu.VMEM((CH, 16), jnp.float32),
            pltpu.SemaphoreType.DMA,
        ],
        compiler_params=pltpu.CompilerParams(use_tc_tiling_on_sc=False),
    )
    return fn(src3, dst3, gd3, ptab, qtab)


def kernel(x, edge_index, edge_attr, W1, att_src1, att_dst1, Wedge1,
           att_edge1, bias1, Wel1, bel1, W2, att_src2, att_dst2, Wedge2,
           att_edge2, bias2, Wel2, bel2, Wc, bc):
    f32 = jnp.float32

    v1 = Wedge1 @ att_edge1
    v2 = Wedge2 @ att_edge2
    U2a = Wel1[:H] @ v2
    U2b = Wel1[H:2 * H] @ v2
    g2w = Wel1[2 * H:] @ v2
    c0 = bel1 @ v2
    Rm = Wel2[2 * H:] @ Wc
    PA2 = Wel2[:H] @ Wc
    PB2 = Wel2[H:2 * H] @ Wc
    PA1 = Wel1[:H] @ Rm
    PB1 = Wel1[H:2 * H] @ Rm
    S8 = Wel1[2 * H:] @ Rm
    cst = bel1 @ Rm + bel2 @ Wc + bc

    zc = jnp.zeros((H, 1), f32)
    A1 = jnp.concatenate(
        [att_src1[:, None], att_dst1[:, None]] + [zc] * 6, axis=1)
    A2h = jnp.concatenate(
        [att_src2[:, None], att_dst2[:, None]] + [zc] * 6, axis=1)
    A2g = jnp.concatenate(
        [jnp.zeros((H, 2), f32), U2a[:, None], U2b[:, None],
         jnp.zeros((H, 4), f32)], axis=1)
    C2 = jnp.zeros((1, 8), f32).at[0, 3].set(c0)
    M16 = jnp.concatenate(
        [S8, v1[:, None], g2w[:, None], jnp.zeros((DE, 6), f32)], axis=1)
    CST = cst[None, :]
    ohs = jnp.concatenate(
        [jnp.eye(3, 16, dtype=f32), jnp.ones((1, 16), f32)], axis=0)

    src = edge_index[0]
    dst = edge_index[1]
    src3 = jnp.pad(src, (0, EPAD - E)).reshape(NW, NCH, CH)
    dst3 = jnp.pad(dst, (0, EPAD - E),
                   constant_values=NN).reshape(NW, NCH, CH)

    def splat(col):
        return jnp.broadcast_to(col[:, None], (NN, 16))

    h1pre, S1 = _tc_prep(x, W1, A1)
    Wbd = jnp.kron(jnp.eye(8, dtype=f32), M16)
    Gd8 = _tc_edge_proj(edge_attr.reshape(E // 8, 128), Wbd)
    gd3 = Gd8.reshape(NGC, CH, 16)

    stab1 = jnp.concatenate([h1pre, splat(S1[:, 0])], axis=1)
    dtab1 = splat(S1[:, 1])
    acc0, acc1 = _sc_sweep(False, 8, src3, dst3, gd3, stab1, dtab1, ohs)
    h1, h2pre, S2 = _tc_combine1(acc0, acc1, h1pre, S1, bias1[None, :],
                                 W2, A2h, A2g, C2)

    stab2 = jnp.concatenate(
        [h2pre, splat(S2[:, 0] + S2[:, 2]), splat(S2[:, 2])], axis=1)
    dtab2 = splat(S2[:, 1] + S2[:, 3])
    acc0b, acc1b = _sc_sweep(True, 9, src3, dst3, gd3, stab2, dtab2, ohs)
    ptab, qtab = _tc_combine2(acc0b, acc1b, h2pre, S2, bias2[None, :],
                              h1, PA2, PA1, PB2, PB1, CST)

    o3 = _sc_outpass(src3, dst3, gd3, ptab, qtab)
    return o3.reshape(E, 16)[:, :8]


# speedup vs baseline: 28.0613x; 1.0042x over previous
import functools

import jax
import jax.numpy as jnp
from jax import lax
from jax.experimental import pallas as pl
from jax.experimental.pallas import tpu as pltpu
from jax.experimental.pallas import tpu_sc as plsc

NN = 10000
E = 320000
F = 128
DE = 16
H = 32
C = 8

NP = 10240
NC = 2
NS = 16
NW = NC * NS
CH = 128
NGC = E // CH
NCH = 79
EPW = NCH * CH
EPAD = NW * EPW
AW = 48
ROWS_PER_TILE = NP // NS


def _prep_body(x_ref, w1_ref, a1_ref, h_ref, s_ref):
    h = jnp.dot(x_ref[...], w1_ref[...], preferred_element_type=jnp.float32)
    h_ref[...] = h
    s_ref[...] = jnp.dot(h, a1_ref[...], preferred_element_type=jnp.float32)


def _tc_prep(x, W1, A1):
    blk = 2000
    grid = NN // blk
    return pl.pallas_call(
        _prep_body,
        grid=(grid,),
        in_specs=[
            pl.BlockSpec((blk, F), lambda i: (i, 0)),
            pl.BlockSpec((F, H), lambda i: (0, 0)),
            pl.BlockSpec((H, 8), lambda i: (0, 0)),
        ],
        out_specs=[
            pl.BlockSpec((blk, H), lambda i: (i, 0)),
            pl.BlockSpec((blk, 8), lambda i: (i, 0)),
        ],
        out_shape=[
            jax.ShapeDtypeStruct((NN, H), jnp.float32),
            jax.ShapeDtypeStruct((NN, 8), jnp.float32),
        ],
    )(x, W1, A1)


def _edge_body(ea_ref, m_ref, o_ref):
    o_ref[...] = jnp.dot(ea_ref[...], m_ref[...],
                         preferred_element_type=jnp.float32)


def _tc_edge_proj(ea8, Wbd):
    rows = E // 8
    blk = 8000
    grid = rows // blk
    return pl.pallas_call(
        _edge_body,
        grid=(grid,),
        in_specs=[
            pl.BlockSpec((blk, 128), lambda i: (i, 0)),
            pl.BlockSpec((128, 128), lambda i: (0, 0)),
        ],
        out_specs=pl.BlockSpec((blk, 128), lambda i: (i, 0)),
        out_shape=jax.ShapeDtypeStruct((rows, 128), jnp.float32),
    )(ea8, Wbd)


def _combine_norm(a0, a1, s, hpre, bias):
    num = a0[:, :H] + a1[:, :H]
    wsum = a0[:, H] + a1[:, H]
    gsum = a0[:, H + 1] + a1[:, H + 1]
    deg = a0[:, H + 2] + a1[:, H + 2]
    gl = jnp.where(deg > 0, gsum / jnp.maximum(deg, 1.0) + s[:, 3], 0.0)
    ln = s[:, 0] + s[:, 1] + gl
    ln = jnp.where(ln >= 0, ln, 0.2 * ln)
    wl = jnp.exp(ln)
    return (num + wl[:, None] * hpre) / (wsum + wl)[:, None] + bias


def _combine1_body(a0_ref, a1_ref, hpre_ref, s_ref, b_ref, w2_ref,
                   a2h_ref, a2g_ref, c2_ref, h1_ref, h2p_ref, s2_ref):
    h1 = _combine_norm(a0_ref[...], a1_ref[...], s_ref[...], hpre_ref[...],
                       b_ref[...])
    h2p = jnp.dot(h1, w2_ref[...], preferred_element_type=jnp.float32)
    s2 = (jnp.dot(h2p, a2h_ref[...], preferred_element_type=jnp.float32)
          + jnp.dot(h1, a2g_ref[...], preferred_element_type=jnp.float32)
          + c2_ref[...])
    h1_ref[...] = h1
    h2p_ref[...] = h2p
    s2_ref[...] = s2


def _tc_combine1(acc0, acc1, h1pre, S1, bias1, W2, A2h, A2g, C2):
    blk = 2000
    grid = NN // blk
    full32 = pl.BlockSpec((H, H), lambda i: (0, 0))
    full328 = pl.BlockSpec((H, 8), lambda i: (0, 0))
    return pl.pallas_call(
        _combine1_body,
        grid=(grid,),
        in_specs=[
            pl.BlockSpec((blk, AW), lambda i: (i, 0)),
            pl.BlockSpec((blk, AW), lambda i: (i, 0)),
            pl.BlockSpec((blk, H), lambda i: (i, 0)),
            pl.BlockSpec((blk, 8), lambda i: (i, 0)),
            pl.BlockSpec((1, H), lambda i: (0, 0)),
            full32, full328, full328,
            pl.BlockSpec((1, 8), lambda i: (0, 0)),
        ],
        out_specs=[
            pl.BlockSpec((blk, H), lambda i: (i, 0)),
            pl.BlockSpec((blk, H), lambda i: (i, 0)),
            pl.BlockSpec((blk, 8), lambda i: (i, 0)),
        ],
        out_shape=[
            jax.ShapeDtypeStruct((NN, H), jnp.float32),
            jax.ShapeDtypeStruct((NN, H), jnp.float32),
            jax.ShapeDtypeStruct((NN, 8), jnp.float32),
        ],
    )(acc0, acc1, h1pre, S1, bias1, W2, A2h, A2g, C2)


def _combine2_body(a0_ref, a1_ref, hpre_ref, s_ref, b_ref, h1_ref,
                   pa2_ref, pa1_ref, pb2_ref, pb1_ref, cst_ref,
                   p_ref, q_ref):
    h2 = _combine_norm(a0_ref[...], a1_ref[...], s_ref[...], hpre_ref[...],
                       b_ref[...])
    h1 = h1_ref[...]
    p = (jnp.dot(h2, pa2_ref[...], preferred_element_type=jnp.float32)
         + jnp.dot(h1, pa1_ref[...], preferred_element_type=jnp.float32))
    q = (jnp.dot(h2, pb2_ref[...], preferred_element_type=jnp.float32)
         + jnp.dot(h1, pb1_ref[...], preferred_element_type=jnp.float32)
         + cst_ref[...])
    z = jnp.zeros_like(p)
    p_ref[...] = jnp.concatenate([p, z], axis=1)
    q_ref[...] = jnp.concatenate([q, z], axis=1)


def _tc_combine2(acc0, acc1, h2pre, S2, bias2, h1, PA2, PA1, PB2, PB1, CST):
    blk = 2000
    grid = NN // blk
    full328 = pl.BlockSpec((H, 8), lambda i: (0, 0))
    return pl.pallas_call(
        _combine2_body,
        grid=(grid,),
        in_specs=[
            pl.BlockSpec((blk, AW), lambda i: (i, 0)),
            pl.BlockSpec((blk, AW), lambda i: (i, 0)),
            pl.BlockSpec((blk, H), lambda i: (i, 0)),
            pl.BlockSpec((blk, 8), lambda i: (i, 0)),
            pl.BlockSpec((1, H), lambda i: (0, 0)),
            pl.BlockSpec((blk, H), lambda i: (i, 0)),
            full328, full328, full328, full328,
            pl.BlockSpec((1, 8), lambda i: (0, 0)),
        ],
        out_specs=[
            pl.BlockSpec((blk, 16), lambda i: (i, 0)),
            pl.BlockSpec((blk, 16), lambda i: (i, 0)),
        ],
        out_shape=[
            jax.ShapeDtypeStruct((NN, 16), jnp.float32),
            jax.ShapeDtypeStruct((NN, 16), jnp.float32),
        ],
    )(acc0, acc1, h2pre, S2, bias2, h1, PA2, PA1, PB2, PB1, CST)


def _sweep_body(use_pab, glane, src_hbm, dst_hbm, gd_hbm, stab_hbm, dtab_hbm,
                ohs_hbm, acc0_hbm, acc1_hbm,
                src_v, dst_v, rows_v, drows_v, ge_v, out_v, ohs_v,
                acc_s, gsem):
    core = lax.axis_index("c")
    sid = lax.axis_index("s")
    wid = sid * NC + core

    pltpu.sync_copy(src_hbm.at[wid], src_v)
    pltpu.sync_copy(dst_hbm.at[wid], dst_v)
    pltpu.sync_copy(ohs_hbm, ohs_v)
    oh0 = ohs_v[0, pl.ds(0, 16)]
    oh1 = ohs_v[1, pl.ds(0, 16)]
    oh2 = ohs_v[2, pl.ds(0, 16)]
    one = ohs_v[3, pl.ds(0, 16)]

    zv = jnp.zeros((16,), jnp.float32)

    def _zbody(e, carry):
        out_v[e, pl.ds(0, 16)] = zv
        out_v[e, pl.ds(16, 16)] = zv
        out_v[e, pl.ds(32, 16)] = zv
        return carry

    lax.fori_loop(0, CH, _zbody, 0)
    for r in range(ROWS_PER_TILE // CH):
        pltpu.sync_copy(out_v,
                        acc_s.at[pl.ds(sid * ROWS_PER_TILE + r * CH, CH)])
    plsc.subcore_barrier()

    def _chunk(j, carry):
        gc = wid * NCH + j

        @pl.when(gc < NGC)
        def _():
            cp1 = pltpu.async_copy(stab_hbm.at[src_v.at[j]], rows_v, gsem)
            cp2 = pltpu.async_copy(dtab_hbm.at[dst_v.at[j]], drows_v, gsem)
            pltpu.sync_copy(gd_hbm.at[gc], ge_v)
            cp1.wait()
            cp2.wait()
            for e in range(CH):
                g = one * ge_v[e, pl.ds(0, 16)][glane]
                a = g
                if use_pab:
                    a = g + rows_v[e, pl.ds(H + 16, 16)]
                l = rows_v[e, pl.ds(H, 16)] + drows_v[e, pl.ds(0, 16)] + g
                l = jnp.maximum(l, 0.2 * l)
                w = jnp.exp(l)
                out_v[e, pl.ds(0, 16)] = rows_v[e, pl.ds(0, 16)] * w
                out_v[e, pl.ds(16, 16)] = rows_v[e, pl.ds(16, 16)] * w
                out_v[e, pl.ds(32, 16)] = oh0 * w + oh1 * a + oh2
            pltpu.sync_copy(out_v, acc_s.at[dst_v.at[j]], add=True)

        return carry

    lax.fori_loop(0, NCH, _chunk, 0)
    plsc.subcore_barrier()

    rows = pl.ds(sid * ROWS_PER_TILE, ROWS_PER_TILE)

    @pl.when(core == 0)
    def _():
        pltpu.sync_copy(acc_s.at[rows], acc0_hbm.at[rows])

    @pl.when(core == 1)
    def _():
        pltpu.sync_copy(acc_s.at[rows], acc1_hbm.at[rows])


def _sc_sweep(use_pab, glane, src3, dst3, gd3, stab, dtab, ohs):
    sw = H + (32 if use_pab else 16)
    mesh = plsc.VectorSubcoreMesh(core_axis_name="c", subcore_axis_name="s")
    fn = pl.kernel(
        functools.partial(_sweep_body, use_pab, glane),
        mesh=mesh,
        out_type=[
            jax.ShapeDtypeStruct((NP, AW), jnp.float32),
            jax.ShapeDtypeStruct((NP, AW), jnp.float32),
        ],
        scratch_types=[
            pltpu.VMEM((NCH, CH), jnp.int32),
            pltpu.VMEM((NCH, CH), jnp.int32),
            pltpu.VMEM((CH, sw), jnp.float32),
            pltpu.VMEM((CH, 16), jnp.float32),
            pltpu.VMEM((CH, 16), jnp.float32),
            pltpu.VMEM((CH, AW), jnp.float32),
            pltpu.VMEM((4, 16), jnp.float32),
            pltpu.VMEM_SHARED((NP, AW), jnp.float32),
            pltpu.SemaphoreType.DMA,
        ],
        compiler_params=pltpu.CompilerParams(use_tc_tiling_on_sc=False),
    )
    return fn(src3, dst3, gd3, stab, dtab, ohs)


def _out_body(src_hbm, dst_hbm, gd_hbm, p_hbm, q_hbm, msk_hbm, rot_hbm,
              o_hbm, src_v, dst_v, t_v, p_v, q_v, o_v, msk_v, rot_v, gsem):
    core = lax.axis_index("c")
    sid = lax.axis_index("s")
    wid = sid * NC + core
    pltpu.sync_copy(src_hbm.at[wid], src_v)
    pltpu.sync_copy(dst_hbm.at[wid], dst_v)
    pltpu.sync_copy(msk_hbm, msk_v)
    pltpu.sync_copy(rot_hbm, rot_v)
    mlo = msk_v[0, pl.ds(0, 16)]
    mhi = msk_v[1, pl.ds(0, 16)]
    rot = rot_v[pl.ds(0, 16)]

    def _chunk(j, carry):
        gc = wid * NCH + j

        @pl.when(gc < NGC)
        def _():
            cp1 = pltpu.async_copy(p_hbm.at[src_v.at[j]], p_v, gsem)
            cp2 = pltpu.async_copy(q_hbm.at[dst_v.at[j]], q_v, gsem)
            pltpu.sync_copy(gd_hbm.at[gc], t_v)
            cp1.wait()
            cp2.wait()

            def _pair(r, c2):
                e0 = 2 * r
                r0 = (t_v[e0, pl.ds(0, 16)] + p_v[e0, pl.ds(0, 16)]
                      + q_v[e0, pl.ds(0, 16)])
                r1 = (t_v[e0 + 1, pl.ds(0, 16)] + p_v[e0 + 1, pl.ds(0, 16)]
                      + q_v[e0 + 1, pl.ds(0, 16)])
                r1r = lax.gather(
                    r1, rot[:, None],
                    lax.GatherDimensionNumbers(
                        offset_dims=(), collapsed_slice_dims=(0,),
                        start_index_map=(0,)),
                    (1,), mode=lax.GatherScatterMode.PROMISE_IN_BOUNDS)
                o_v[r, pl.ds(0, 16)] = r0 * mlo + r1r * mhi
                return c2

            lax.fori_loop(0, CH // 2, _pair, 0)
            pltpu.sync_copy(o_v, o_hbm.at[gc])

        return carry

    lax.fori_loop(0, NCH, _chunk, 0)


def _sc_outpass(src3, dst3, gd3, ptab, qtab, msk, rot):
    mesh = plsc.VectorSubcoreMesh(core_axis_name="c", subcore_axis_name="s")
    fn = pl.kernel(
        _out_body,
        mesh=mesh,
        out_type=jax.ShapeDtypeStruct((NGC, CH // 2, 16), jnp.float32),
        scratch_types=[
            pltpu.VMEM((NCH, CH), jnp.int32),
            pltpu.VMEM((NCH, CH), jnp.int32),
            pltpu.VMEM((CH, 16), jnp.float32),
            pltpu.VMEM((CH, 16), jnp.float32),
            pltpu.VMEM((CH, 16), jnp.float32),
            pltpu.VMEM((CH // 2, 16), jnp.float32),
            pltpu.VMEM((2, 16), jnp.float32),
            pltpu.VMEM((16,), jnp.int32),
            pltpu.SemaphoreType.DMA,
        ],
        compiler_params=pltpu.CompilerParams(use_tc_tiling_on_sc=False),
    )
    return fn(src3, dst3, gd3, ptab, qtab, msk, rot)


def kernel(x, edge_index, edge_attr, W1, att_src1, att_dst1, Wedge1,
           att_edge1, bias1, Wel1, bel1, W2, att_src2, att_dst2, Wedge2,
           att_edge2, bias2, Wel2, bel2, Wc, bc):
    f32 = jnp.float32

    v1 = Wedge1 @ att_edge1
    v2 = Wedge2 @ att_edge2
    U2a = Wel1[:H] @ v2
    U2b = Wel1[H:2 * H] @ v2
    g2w = Wel1[2 * H:] @ v2
    c0 = bel1 @ v2
    Rm = Wel2[2 * H:] @ Wc
    PA2 = Wel2[:H] @ Wc
    PB2 = Wel2[H:2 * H] @ Wc
    PA1 = Wel1[:H] @ Rm
    PB1 = Wel1[H:2 * H] @ Rm
    S8 = Wel1[2 * H:] @ Rm
    cst = bel1 @ Rm + bel2 @ Wc + bc

    zc = jnp.zeros((H, 1), f32)
    A1 = jnp.concatenate(
        [att_src1[:, None], att_dst1[:, None]] + [zc] * 6, axis=1)
    A2h = jnp.concatenate(
        [att_src2[:, None], att_dst2[:, None]] + [zc] * 6, axis=1)
    A2g = jnp.concatenate(
        [jnp.zeros((H, 2), f32), U2a[:, None], U2b[:, None],
         jnp.zeros((H, 4), f32)], axis=1)
    C2 = jnp.zeros((1, 8), f32).at[0, 3].set(c0)
    M16 = jnp.concatenate(
        [S8, v1[:, None], g2w[:, None], jnp.zeros((DE, 6), f32)], axis=1)
    CST = cst[None, :]
    ohs = jnp.concatenate(
        [jnp.eye(3, 16, dtype=f32), jnp.ones((1, 16), f32)], axis=0)

    src = edge_index[0]
    dst = edge_index[1]
    src3 = jnp.pad(src, (0, EPAD - E)).reshape(NW, NCH, CH)
    dst3 = jnp.pad(dst, (0, EPAD - E),
                   constant_values=NN).reshape(NW, NCH, CH)

    def splat(col):
        return jnp.broadcast_to(col[:, None], (NN, 16))

    h1pre, S1 = _tc_prep(x, W1, A1)
    Wbd = jnp.kron(jnp.eye(8, dtype=f32), M16)
    Gd8 = _tc_edge_proj(edge_attr.reshape(E // 8, 128), Wbd)
    gd3 = Gd8.reshape(NGC, CH, 16)

    stab1 = jnp.concatenate([h1pre, splat(S1[:, 0])], axis=1)
    dtab1 = splat(S1[:, 1])
    acc0, acc1 = _sc_sweep(False, 8, src3, dst3, gd3, stab1, dtab1, ohs)
    h1, h2pre, S2 = _tc_combine1(acc0, acc1, h1pre, S1, bias1[None, :],
                                 W2, A2h, A2g, C2)

    stab2 = jnp.concatenate(
        [h2pre, splat(S2[:, 0] + S2[:, 2]), splat(S2[:, 2])], axis=1)
    dtab2 = splat(S2[:, 1] + S2[:, 3])
    acc0b, acc1b = _sc_sweep(True, 9, src3, dst3, gd3, stab2, dtab2, ohs)
    ptab, qtab = _tc_combine2(acc0b, acc1b, h2pre, S2, bias2[None, :],
                              h1, PA2, PA1, PB2, PB1, CST)

    msk = jnp.concatenate(
        [jnp.ones((1, 8), f32), jnp.zeros((1, 8), f32)], axis=1)
    msk = jnp.concatenate([msk, 1.0 - msk], axis=0)
    rot = (jnp.arange(16, dtype=jnp.int32) + 8) % 16
    o3 = _sc_outpass(src3, dst3, gd3, ptab, qtab, msk, rot)
    return o3.reshape(E, 8)
